# initial kernel scaffold (unmeasured)
import jax
import jax.numpy as jnp
from jax import lax
from jax.experimental import pallas as pl
from jax.experimental.pallas import tpu as pltpu

N_DEV = 4
M = 4096
KSH = 1024
NCOL = 8192
CH = M // N_DEV
TN = 2048
NT = NCOL // TN


def kernel(x, w_mat):
    def body(x_ref, w_ref, out_ref, comm, gat, tin, qt, amax_ref,
             rs_send, rs_recv, ax_send, ax_recv, ag_send, ag_recv, cp_sem):
        d = lax.axis_index("i")
        left = lax.rem(d + N_DEV - 1, N_DEV)
        right = lax.rem(d + 1, N_DEV)

        barrier = pltpu.get_barrier_semaphore()
        for nbr in (left, right):
            pl.semaphore_signal(barrier, inc=1, device_id=(nbr,),
                                device_id_type=pl.DeviceIdType.MESH)
        pl.semaphore_wait(barrier, 2)

        def partial_tile(c, j):
            xc = x_ref[pl.ds(c * CH, CH), :]
            return jnp.dot(xc, w_ref[:, j * TN:(j + 1) * TN],
                           preferred_element_type=jnp.float32)

        def copy(src, dst):
            cp = pltpu.make_async_copy(src, dst, cp_sem)
            cp.start()
            cp.wait()

        for j in range(NT):
            tin[...] = partial_tile(d, j).astype(jnp.bfloat16)
            copy(tin, comm.at[0, j])

        amax_val = jnp.float32(0.0)
        for h in range(N_DEV - 1):
            rdma = pltpu.make_async_remote_copy(
                src_ref=comm.at[h],
                dst_ref=comm.at[h + 1],
                send_sem=rs_send.at[h],
                recv_sem=rs_recv.at[h],
                device_id=(right,),
                device_id_type=pl.DeviceIdType.MESH,
            )
            rdma.start()
            rdma.wait()
            c_in = lax.rem(d + 2 * N_DEV - 1 - h, N_DEV)
            last = h == N_DEV - 2
            for j in range(NT):
                copy(comm.at[h + 1, j], tin)
                acc = tin[...].astype(jnp.float32) + partial_tile(c_in, j)
                if last:
                    acc = jnp.maximum(acc, 0.0)
                    amax_val = jnp.maximum(amax_val, jnp.max(acc))
                tin[...] = acc.astype(jnp.bfloat16)
                copy(tin, comm.at[h + 1, j])

        amax_ref[pl.ds(d, 1)] = jnp.full((1, 8, 128), amax_val, jnp.float32)
        sends = []
        for k in range(1, N_DEV):
            peer = lax.rem(d + k, N_DEV)
            r = pltpu.make_async_remote_copy(
                src_ref=amax_ref.at[pl.ds(d, 1)],
                dst_ref=amax_ref.at[pl.ds(d, 1)],
                send_sem=ax_send.at[k - 1],
                recv_sem=ax_recv.at[k - 1],
                device_id=(peer,),
                device_id_type=pl.DeviceIdType.MESH,
            )
            r.start()
            sends.append(r)
        for r in sends:
            r.wait_send()
        for k in range(1, N_DEV):
            src = lax.rem(d + N_DEV - k, N_DEV)
            r = pltpu.make_async_remote_copy(
                src_ref=amax_ref.at[pl.ds(d, 1)],
                dst_ref=amax_ref.at[pl.ds(src, 1)],
                send_sem=ax_send.at[k - 1],
                recv_sem=ax_recv.at[k - 1],
                device_id=(d,),
                device_id_type=pl.DeviceIdType.MESH,
            )
            r.wait_recv()
        amax_g = jnp.maximum(jnp.max(amax_ref[...]), jnp.float32(1e-30))
        scale = amax_g / 448.0
        inv_scale = 448.0 / amax_g

        c_own = lax.rem(d + 1, N_DEV)
        for j in range(NT):
            copy(comm.at[N_DEV - 1, j], tin)
            v = jnp.minimum(tin[...].astype(jnp.float32) * inv_scale, 448.0)
            qt[...] = v.astype(jnp.float8_e4m3fn)
            copy(qt, gat.at[c_own, j])

        for h in range(N_DEV - 1):
            c_s = lax.rem(d + 1 + N_DEV - h, N_DEV)
            r = pltpu.make_async_remote_copy(
                src_ref=gat.at[c_s],
                dst_ref=gat.at[c_s],
                send_sem=ag_send.at[h],
                recv_sem=ag_recv.at[h],
                device_id=(right,),
                device_id_type=pl.DeviceIdType.MESH,
            )
            r.start()
            r.wait()

        for c in range(N_DEV):
            for j in range(NT):
                copy(gat.at[c, j], qt)
                tin[...] = (qt[...].astype(jnp.float32) * scale).astype(
                    jnp.bfloat16)
                copy(tin, out_ref.at[pl.ds(c * CH, CH), pl.ds(j * TN, TN)])

    out_shape = jax.ShapeDtypeStruct((M, NCOL), jnp.bfloat16)
    return pl.pallas_call(
        body,
        out_shape=out_shape,
        in_specs=[
            pl.BlockSpec(memory_space=pltpu.MemorySpace.VMEM),
            pl.BlockSpec(memory_space=pltpu.MemorySpace.VMEM),
        ],
        out_specs=pl.BlockSpec(memory_space=pltpu.MemorySpace.HBM),
        scratch_shapes=[
            pltpu.MemorySpace.HBM((N_DEV, NT, CH, TN), jnp.bfloat16),
            pltpu.MemorySpace.HBM((N_DEV, NT, CH, TN), jnp.float8_e4m3fn),
            pltpu.VMEM((CH, TN), jnp.bfloat16),
            pltpu.VMEM((CH, TN), jnp.float8_e4m3fn),
            pltpu.VMEM((N_DEV, 8, 128), jnp.float32),
            pltpu.SemaphoreType.DMA((N_DEV - 1,)),
            pltpu.SemaphoreType.DMA((N_DEV - 1,)),
            pltpu.SemaphoreType.DMA((N_DEV - 1,)),
            pltpu.SemaphoreType.DMA((N_DEV - 1,)),
            pltpu.SemaphoreType.DMA((N_DEV - 1,)),
            pltpu.SemaphoreType.DMA((N_DEV - 1,)),
            pltpu.SemaphoreType.DMA,
        ],
        compiler_params=pltpu.CompilerParams(collective_id=0),
    )(x, w_mat)


# baseline (device time: 1126918 ns/iter reference)
import jax
import jax.numpy as jnp
from jax import lax
from jax.experimental import pallas as pl
from jax.experimental.pallas import tpu as pltpu

N_DEV = 4
M = 4096
KSH = 1024
NCOL = 8192
CH = M // N_DEV
TN = 2048
NT = NCOL // TN


def kernel(x, w_mat):
    x = x.astype(jnp.bfloat16)
    w_mat = w_mat.astype(jnp.bfloat16)

    def body(x_ref, w_ref, out_ref, comm, gat, tin, qt, amax_ref,
             rs_send, rs_recv, ax_send, ax_recv, ag_send, ag_recv, cp_sem):
        d = lax.axis_index("i")
        left = lax.rem(d + N_DEV - 1, N_DEV)
        right = lax.rem(d + 1, N_DEV)

        barrier = pltpu.get_barrier_semaphore()
        for nbr in (left, right):
            pl.semaphore_signal(barrier, inc=1, device_id=(nbr,),
                                device_id_type=pl.DeviceIdType.MESH)
        pl.semaphore_wait(barrier, 2)

        def partial_tile(c, j):
            xc = x_ref[pl.ds(c * CH, CH), :]
            return jnp.dot(xc, w_ref[:, j * TN:(j + 1) * TN],
                           preferred_element_type=jnp.float32)

        def copy(src, dst):
            cp = pltpu.make_async_copy(src, dst, cp_sem)
            cp.start()
            cp.wait()

        for j in range(NT):
            tin[...] = partial_tile(d, j).astype(jnp.bfloat16)
            copy(tin, comm.at[0, j])

        amax_val = jnp.float32(0.0)
        for h in range(N_DEV - 1):
            rdma = pltpu.make_async_remote_copy(
                src_ref=comm.at[h],
                dst_ref=comm.at[h + 1],
                send_sem=rs_send.at[h],
                recv_sem=rs_recv.at[h],
                device_id=(right,),
                device_id_type=pl.DeviceIdType.MESH,
            )
            rdma.start()
            rdma.wait()
            c_in = lax.rem(d + 2 * N_DEV - 1 - h, N_DEV)
            last = h == N_DEV - 2
            for j in range(NT):
                copy(comm.at[h + 1, j], tin)
                acc = tin[...].astype(jnp.float32) + partial_tile(c_in, j)
                if last:
                    acc = jnp.maximum(acc, 0.0)
                    amax_val = jnp.maximum(amax_val, jnp.max(acc))
                tin[...] = acc.astype(jnp.bfloat16)
                copy(tin, comm.at[h + 1, j])

        amax_ref[pl.ds(d, 1)] = jnp.full((1, 8, 128), amax_val, jnp.float32)
        sends = []
        for k in range(1, N_DEV):
            peer = lax.rem(d + k, N_DEV)
            r = pltpu.make_async_remote_copy(
                src_ref=amax_ref.at[pl.ds(d, 1)],
                dst_ref=amax_ref.at[pl.ds(d, 1)],
                send_sem=ax_send.at[k - 1],
                recv_sem=ax_recv.at[k - 1],
                device_id=(peer,),
                device_id_type=pl.DeviceIdType.MESH,
            )
            r.start()
            sends.append(r)
        for r in sends:
            r.wait_send()
        for k in range(1, N_DEV):
            src = lax.rem(d + N_DEV - k, N_DEV)
            r = pltpu.make_async_remote_copy(
                src_ref=amax_ref.at[pl.ds(d, 1)],
                dst_ref=amax_ref.at[pl.ds(src, 1)],
                send_sem=ax_send.at[k - 1],
                recv_sem=ax_recv.at[k - 1],
                device_id=(d,),
                device_id_type=pl.DeviceIdType.MESH,
            )
            r.wait_recv()
        amax_g = jnp.maximum(jnp.max(amax_ref[...]), jnp.float32(1e-30))
        scale = amax_g / 448.0
        inv_scale = 448.0 / amax_g

        c_own = lax.rem(d + 1, N_DEV)
        for j in range(NT):
            copy(comm.at[N_DEV - 1, j], tin)
            v = jnp.minimum(tin[...].astype(jnp.float32) * inv_scale, 448.0)
            qt[...] = v.astype(jnp.float8_e4m3fn)
            copy(qt, gat.at[c_own, j])

        for h in range(N_DEV - 1):
            c_s = lax.rem(d + 1 + N_DEV - h, N_DEV)
            r = pltpu.make_async_remote_copy(
                src_ref=gat.at[c_s],
                dst_ref=gat.at[c_s],
                send_sem=ag_send.at[h],
                recv_sem=ag_recv.at[h],
                device_id=(right,),
                device_id_type=pl.DeviceIdType.MESH,
            )
            r.start()
            r.wait()

        for c in range(N_DEV):
            for j in range(NT):
                copy(gat.at[c, j], qt)
                tin[...] = (qt[...].astype(jnp.float32) * scale).astype(
                    jnp.bfloat16)
                copy(tin, out_ref.at[pl.ds(c * CH, CH), pl.ds(j * TN, TN)])

    out_shapes = [
        jax.ShapeDtypeStruct((M, NCOL), jnp.bfloat16),
        jax.ShapeDtypeStruct((N_DEV, NT, CH, TN), jnp.bfloat16),
        jax.ShapeDtypeStruct((N_DEV, NT, CH, TN), jnp.float8_e4m3fn),
    ]
    out, _, _ = pl.pallas_call(
        body,
        out_shape=out_shapes,
        in_specs=[
            pl.BlockSpec(memory_space=pltpu.MemorySpace.VMEM),
            pl.BlockSpec(memory_space=pltpu.MemorySpace.VMEM),
        ],
        out_specs=[
            pl.BlockSpec(memory_space=pltpu.MemorySpace.HBM),
            pl.BlockSpec(memory_space=pltpu.MemorySpace.HBM),
            pl.BlockSpec(memory_space=pltpu.MemorySpace.HBM),
        ],
        scratch_shapes=[
            pltpu.VMEM((CH, TN), jnp.bfloat16),
            pltpu.VMEM((CH, TN), jnp.float8_e4m3fn),
            pltpu.VMEM((N_DEV, 8, 128), jnp.float32),
            pltpu.SemaphoreType.DMA((N_DEV - 1,)),
            pltpu.SemaphoreType.DMA((N_DEV - 1,)),
            pltpu.SemaphoreType.DMA((N_DEV - 1,)),
            pltpu.SemaphoreType.DMA((N_DEV - 1,)),
            pltpu.SemaphoreType.DMA((N_DEV - 1,)),
            pltpu.SemaphoreType.DMA((N_DEV - 1,)),
            pltpu.SemaphoreType.DMA,
        ],
        compiler_params=pltpu.CompilerParams(
            collective_id=0, vmem_limit_bytes=64 * 1024 * 1024),
    )(x, w_mat)
    return out


# device time: 543434 ns/iter; 2.0737x vs baseline; 2.0737x over previous
import jax
import jax.numpy as jnp
from jax import lax
from jax.experimental import pallas as pl
from jax.experimental.pallas import tpu as pltpu

N_DEV = 4
M = 4096
KSH = 1024
NCOL = 8192
CH = M // N_DEV
TN = 2048
NT = NCOL // TN
TILE_ORDER = (0, 2, 1, 3)


def kernel(x, w_mat):
    x = x.astype(jnp.bfloat16)
    w_mat = w_mat.astype(jnp.bfloat16)

    def body(x_ref, w_ref, out_ref, comm, gat, tin, qt, amax_ref,
             rs_send, rs_recv, ax_send, ax_recv, ag_send, ag_recv, cp_sem):
        d = lax.axis_index("i")
        left = lax.rem(d + N_DEV - 1, N_DEV)
        right = lax.rem(d + 1, N_DEV)

        def ring_nbr(j):
            return right if j < NT // 2 else left

        barrier = pltpu.get_barrier_semaphore()
        for nbr in (left, right):
            pl.semaphore_signal(barrier, inc=1, device_id=(nbr,),
                                device_id_type=pl.DeviceIdType.MESH)
        pl.semaphore_wait(barrier, 2)

        def partial_tile(c, j):
            xc = x_ref[pl.ds(c * CH, CH), :]
            return jnp.dot(xc, w_ref[:, j * TN:(j + 1) * TN],
                           preferred_element_type=jnp.float32)

        def copy(src, dst):
            cp = pltpu.make_async_copy(src, dst, cp_sem)
            cp.start()
            cp.wait()

        def rs_rdma(h, j):
            return pltpu.make_async_remote_copy(
                src_ref=comm.at[h, j],
                dst_ref=comm.at[h + 1, j],
                send_sem=rs_send.at[h, j],
                recv_sem=rs_recv.at[h, j],
                device_id=(ring_nbr(j),),
                device_id_type=pl.DeviceIdType.MESH,
            )

        hop_rdmas = []
        for j in TILE_ORDER:
            tin[...] = partial_tile(d, j).astype(jnp.bfloat16)
            copy(tin, comm.at[0, j])
            r = rs_rdma(0, j)
            r.start()
            hop_rdmas.append(r)

        amax_val = jnp.float32(0.0)
        for h in range(N_DEV - 1):
            last = h == N_DEV - 2
            next_rdmas = []
            for j in TILE_ORDER:
                if j < NT // 2:
                    c_in = lax.rem(d + 2 * N_DEV - 1 - h, N_DEV)
                else:
                    c_in = lax.rem(d + 1 + h, N_DEV)
                hop_rdmas[TILE_ORDER.index(j)].wait_recv()
                copy(comm.at[h + 1, j], tin)
                acc = tin[...].astype(jnp.float32) + partial_tile(c_in, j)
                if last:
                    acc = jnp.maximum(acc, 0.0)
                    amax_val = jnp.maximum(amax_val, jnp.max(acc))
                tin[...] = acc.astype(jnp.bfloat16)
                copy(tin, comm.at[h + 1, j])
                if not last:
                    r = rs_rdma(h + 1, j)
                    r.start()
                    next_rdmas.append(r)
            for r in hop_rdmas:
                r.wait_send()
            hop_rdmas = next_rdmas

        amax_ref[pl.ds(d, 1)] = jnp.full((1, 8, 128), amax_val, jnp.float32)
        sends = []
        for k in range(1, N_DEV):
            peer = lax.rem(d + k, N_DEV)
            r = pltpu.make_async_remote_copy(
                src_ref=amax_ref.at[pl.ds(d, 1)],
                dst_ref=amax_ref.at[pl.ds(d, 1)],
                send_sem=ax_send.at[k - 1],
                recv_sem=ax_recv.at[k - 1],
                device_id=(peer,),
                device_id_type=pl.DeviceIdType.MESH,
            )
            r.start()
            sends.append(r)
        for r in sends:
            r.wait_send()
        for k in range(1, N_DEV):
            src = lax.rem(d + N_DEV - k, N_DEV)
            r = pltpu.make_async_remote_copy(
                src_ref=amax_ref.at[pl.ds(d, 1)],
                dst_ref=amax_ref.at[pl.ds(src, 1)],
                send_sem=ax_send.at[k - 1],
                recv_sem=ax_recv.at[k - 1],
                device_id=(d,),
                device_id_type=pl.DeviceIdType.MESH,
            )
            r.wait_recv()
        amax_g = jnp.maximum(jnp.max(amax_ref[...]), jnp.float32(1e-30))
        scale = amax_g / 448.0
        inv_scale = 448.0 / amax_g

        own_r = lax.rem(d + 1, N_DEV)
        own_l = lax.rem(d + 3, N_DEV)

        def owned_chunk(j):
            return own_r if j < NT // 2 else own_l

        def ag_chunk(h, j):
            if j < NT // 2:
                return lax.rem(d + 1 + N_DEV - h, N_DEV)
            return lax.rem(d + 3 + h, N_DEV)

        def dequant_to_out(c, j):
            copy(gat.at[c, j], qt)
            tin[...] = (qt[...].astype(jnp.float32) * scale).astype(
                jnp.bfloat16)
            copy(tin, out_ref.at[pl.ds(c * CH, CH), pl.ds(j * TN, TN)])

        def ag_rdma(h, j):
            c = ag_chunk(h, j)
            return pltpu.make_async_remote_copy(
                src_ref=gat.at[c, j],
                dst_ref=gat.at[c, j],
                send_sem=ag_send.at[h, j],
                recv_sem=ag_recv.at[h, j],
                device_id=(ring_nbr(j),),
                device_id_type=pl.DeviceIdType.MESH,
            )

        hop_rdmas = []
        for j in TILE_ORDER:
            c = owned_chunk(j)
            copy(comm.at[N_DEV - 1, j], tin)
            v = jnp.minimum(tin[...].astype(jnp.float32) * inv_scale, 448.0)
            qt[...] = v.astype(jnp.float8_e4m3fn)
            copy(qt, gat.at[c, j])
            r = ag_rdma(0, j)
            r.start()
            hop_rdmas.append(r)

        for j in TILE_ORDER:
            dequant_to_out(owned_chunk(j), j)

        for h in range(N_DEV - 1):
            last = h == N_DEV - 2
            next_rdmas = []
            for j in TILE_ORDER:
                hop_rdmas[TILE_ORDER.index(j)].wait_recv()
                if not last:
                    r = ag_rdma(h + 1, j)
                    r.start()
                    next_rdmas.append(r)
            for j in TILE_ORDER:
                dequant_to_out(ag_chunk(h + 1, j), j)
            for r in hop_rdmas:
                r.wait_send()
            hop_rdmas = next_rdmas

    out_shapes = [
        jax.ShapeDtypeStruct((M, NCOL), jnp.bfloat16),
        jax.ShapeDtypeStruct((N_DEV, NT, CH, TN), jnp.bfloat16),
        jax.ShapeDtypeStruct((N_DEV, NT, CH, TN), jnp.float8_e4m3fn),
    ]
    out, _, _ = pl.pallas_call(
        body,
        out_shape=out_shapes,
        in_specs=[
            pl.BlockSpec(memory_space=pltpu.MemorySpace.VMEM),
            pl.BlockSpec(memory_space=pltpu.MemorySpace.VMEM),
        ],
        out_specs=[
            pl.BlockSpec(memory_space=pltpu.MemorySpace.HBM),
            pl.BlockSpec(memory_space=pltpu.MemorySpace.HBM),
            pl.BlockSpec(memory_space=pltpu.MemorySpace.HBM),
        ],
        scratch_shapes=[
            pltpu.VMEM((CH, TN), jnp.bfloat16),
            pltpu.VMEM((CH, TN), jnp.float8_e4m3fn),
            pltpu.VMEM((N_DEV, 8, 128), jnp.float32),
            pltpu.SemaphoreType.DMA((N_DEV - 1, NT)),
            pltpu.SemaphoreType.DMA((N_DEV - 1, NT)),
            pltpu.SemaphoreType.DMA((N_DEV - 1,)),
            pltpu.SemaphoreType.DMA((N_DEV - 1,)),
            pltpu.SemaphoreType.DMA((N_DEV - 1, NT)),
            pltpu.SemaphoreType.DMA((N_DEV - 1, NT)),
            pltpu.SemaphoreType.DMA,
        ],
        compiler_params=pltpu.CompilerParams(
            collective_id=0, vmem_limit_bytes=64 * 1024 * 1024),
    )(x, w_mat)
    return out


# device time: 534906 ns/iter; 2.1068x vs baseline; 1.0159x over previous
import jax
import jax.numpy as jnp
from jax import lax
from jax.experimental import pallas as pl
from jax.experimental.pallas import tpu as pltpu

N_DEV = 4
M = 4096
KSH = 1024
NCOL = 8192
CH = M // N_DEV
TN = 2048
NT = NCOL // TN
TILE_ORDER = (0, 2, 1, 3)


def kernel(x, w_mat):
    x = x.astype(jnp.bfloat16)
    w_mat = w_mat.astype(jnp.bfloat16)

    def body(x_ref, w_ref, out_ref, comm, gat, sbuf, tin, qt, amax_ref,
             rs_send, rs_recv, ax_send, ax_recv, ag_send, ag_recv, cp_sem):
        d = lax.axis_index("i")
        left = lax.rem(d + N_DEV - 1, N_DEV)
        right = lax.rem(d + 1, N_DEV)

        def ring_nbr(j):
            return right if j < NT // 2 else left

        barrier = pltpu.get_barrier_semaphore()
        for nbr in (left, right):
            pl.semaphore_signal(barrier, inc=1, device_id=(nbr,),
                                device_id_type=pl.DeviceIdType.MESH)
        pl.semaphore_wait(barrier, 2)

        def partial_tile(c, j):
            xc = x_ref[pl.ds(c * CH, CH), :]
            return jnp.dot(xc, w_ref[:, j * TN:(j + 1) * TN],
                           preferred_element_type=jnp.float32)

        def copy(src, dst):
            cp = pltpu.make_async_copy(src, dst, cp_sem)
            cp.start()
            cp.wait()

        def rs_rdma(h, j):
            return pltpu.make_async_remote_copy(
                src_ref=sbuf.at[j],
                dst_ref=comm.at[h, j],
                send_sem=rs_send.at[h, j],
                recv_sem=rs_recv.at[h, j],
                device_id=(ring_nbr(j),),
                device_id_type=pl.DeviceIdType.MESH,
            )

        hop_rdmas = []
        for j in TILE_ORDER:
            sbuf[j] = partial_tile(d, j).astype(jnp.bfloat16)
            r = rs_rdma(0, j)
            r.start()
            hop_rdmas.append(r)

        amax_val = jnp.float32(0.0)
        for h in range(N_DEV - 1):
            last = h == N_DEV - 2
            next_rdmas = []
            for i, j in enumerate(TILE_ORDER):
                if j < NT // 2:
                    c_in = lax.rem(d + 2 * N_DEV - 1 - h, N_DEV)
                else:
                    c_in = lax.rem(d + 1 + h, N_DEV)
                hop_rdmas[i].wait()
                copy(comm.at[h, j], tin)
                acc = tin[...].astype(jnp.float32) + partial_tile(c_in, j)
                if last:
                    acc = jnp.maximum(acc, 0.0)
                    amax_val = jnp.maximum(amax_val, jnp.max(acc))
                sbuf[j] = acc.astype(jnp.bfloat16)
                if not last:
                    r = rs_rdma(h + 1, j)
                    r.start()
                    next_rdmas.append(r)
            hop_rdmas = next_rdmas

        amax_ref[pl.ds(d, 1)] = jnp.full((1, 8, 128), amax_val, jnp.float32)
        sends = []
        for k in range(1, N_DEV):
            peer = lax.rem(d + k, N_DEV)
            r = pltpu.make_async_remote_copy(
                src_ref=amax_ref.at[pl.ds(d, 1)],
                dst_ref=amax_ref.at[pl.ds(d, 1)],
                send_sem=ax_send.at[k - 1],
                recv_sem=ax_recv.at[k - 1],
                device_id=(peer,),
                device_id_type=pl.DeviceIdType.MESH,
            )
            r.start()
            sends.append(r)
        for r in sends:
            r.wait_send()
        for k in range(1, N_DEV):
            src = lax.rem(d + N_DEV - k, N_DEV)
            r = pltpu.make_async_remote_copy(
                src_ref=amax_ref.at[pl.ds(d, 1)],
                dst_ref=amax_ref.at[pl.ds(src, 1)],
                send_sem=ax_send.at[k - 1],
                recv_sem=ax_recv.at[k - 1],
                device_id=(d,),
                device_id_type=pl.DeviceIdType.MESH,
            )
            r.wait_recv()
        amax_g = jnp.maximum(jnp.max(amax_ref[...]), jnp.float32(1e-30))
        scale = amax_g / 448.0
        inv_scale = 448.0 / amax_g

        own_r = lax.rem(d + 1, N_DEV)
        own_l = lax.rem(d + 3, N_DEV)

        def owned_chunk(j):
            return own_r if j < NT // 2 else own_l

        def ag_chunk(h, j):
            if j < NT // 2:
                return lax.rem(d + 1 + N_DEV - h, N_DEV)
            return lax.rem(d + 3 + h, N_DEV)

        def dequant_to_out(c, j):
            copy(gat.at[c, j], qt)
            tin[...] = (qt[...].astype(jnp.float32) * scale).astype(
                jnp.bfloat16)
            copy(tin, out_ref.at[pl.ds(c * CH, CH), pl.ds(j * TN, TN)])

        def ag_rdma(h, j):
            c = ag_chunk(h, j)
            return pltpu.make_async_remote_copy(
                src_ref=gat.at[c, j],
                dst_ref=gat.at[c, j],
                send_sem=ag_send.at[h, j],
                recv_sem=ag_recv.at[h, j],
                device_id=(ring_nbr(j),),
                device_id_type=pl.DeviceIdType.MESH,
            )

        hop_rdmas = []
        for j in TILE_ORDER:
            c = owned_chunk(j)
            v = jnp.minimum(sbuf[j].astype(jnp.float32) * inv_scale, 448.0)
            qt[...] = v.astype(jnp.float8_e4m3fn)
            copy(qt, gat.at[c, j])
            r = ag_rdma(0, j)
            r.start()
            hop_rdmas.append(r)

        for j in TILE_ORDER:
            dequant_to_out(owned_chunk(j), j)

        for h in range(N_DEV - 1):
            last = h == N_DEV - 2
            next_rdmas = []
            for j in TILE_ORDER:
                hop_rdmas[TILE_ORDER.index(j)].wait_recv()
                if not last:
                    r = ag_rdma(h + 1, j)
                    r.start()
                    next_rdmas.append(r)
            for j in TILE_ORDER:
                dequant_to_out(ag_chunk(h + 1, j), j)
            for r in hop_rdmas:
                r.wait_send()
            hop_rdmas = next_rdmas

    out_shapes = [
        jax.ShapeDtypeStruct((M, NCOL), jnp.bfloat16),
        jax.ShapeDtypeStruct((N_DEV - 1, NT, CH, TN), jnp.bfloat16),
        jax.ShapeDtypeStruct((N_DEV, NT, CH, TN), jnp.float8_e4m3fn),
    ]
    out, _, _ = pl.pallas_call(
        body,
        out_shape=out_shapes,
        in_specs=[
            pl.BlockSpec(memory_space=pltpu.MemorySpace.VMEM),
            pl.BlockSpec(memory_space=pltpu.MemorySpace.VMEM),
        ],
        out_specs=[
            pl.BlockSpec(memory_space=pltpu.MemorySpace.HBM),
            pl.BlockSpec(memory_space=pltpu.MemorySpace.HBM),
            pl.BlockSpec(memory_space=pltpu.MemorySpace.HBM),
        ],
        scratch_shapes=[
            pltpu.VMEM((NT, CH, TN), jnp.bfloat16),
            pltpu.VMEM((CH, TN), jnp.bfloat16),
            pltpu.VMEM((CH, TN), jnp.float8_e4m3fn),
            pltpu.VMEM((N_DEV, 8, 128), jnp.float32),
            pltpu.SemaphoreType.DMA((N_DEV - 1, NT)),
            pltpu.SemaphoreType.DMA((N_DEV - 1, NT)),
            pltpu.SemaphoreType.DMA((N_DEV - 1,)),
            pltpu.SemaphoreType.DMA((N_DEV - 1,)),
            pltpu.SemaphoreType.DMA((N_DEV - 1, NT)),
            pltpu.SemaphoreType.DMA((N_DEV - 1, NT)),
            pltpu.SemaphoreType.DMA,
        ],
        compiler_params=pltpu.CompilerParams(
            collective_id=0, vmem_limit_bytes=64 * 1024 * 1024),
    )(x, w_mat)
    return out


# device time: 530894 ns/iter; 2.1227x vs baseline; 1.0076x over previous
import jax
import jax.numpy as jnp
from jax import lax
from jax.experimental import pallas as pl
from jax.experimental.pallas import tpu as pltpu

N_DEV = 4
M = 4096
KSH = 1024
NCOL = 8192
CH = M // N_DEV
TN = 1024
NT = NCOL // TN
TILE_ORDER = (0, 4, 1, 5, 2, 6, 3, 7)


def kernel(x, w_mat):
    x = x.astype(jnp.bfloat16)
    w_mat = w_mat.astype(jnp.bfloat16)

    def body(x_ref, w_ref, out_ref, comm, gat, sbuf, tin, qt, amax_ref,
             rs_send, rs_recv, ax_send, ax_recv, ag_send, ag_recv, cp_sem):
        d = lax.axis_index("i")
        left = lax.rem(d + N_DEV - 1, N_DEV)
        right = lax.rem(d + 1, N_DEV)

        def ring_nbr(j):
            return right if j < NT // 2 else left

        barrier = pltpu.get_barrier_semaphore()
        for nbr in (left, right):
            pl.semaphore_signal(barrier, inc=1, device_id=(nbr,),
                                device_id_type=pl.DeviceIdType.MESH)
        pl.semaphore_wait(barrier, 2)

        def partial_tile(c, j):
            xc = x_ref[pl.ds(c * CH, CH), :]
            return jnp.dot(xc, w_ref[:, j * TN:(j + 1) * TN],
                           preferred_element_type=jnp.float32)

        def copy(src, dst):
            cp = pltpu.make_async_copy(src, dst, cp_sem)
            cp.start()
            cp.wait()

        def rs_rdma(h, j):
            return pltpu.make_async_remote_copy(
                src_ref=sbuf.at[j],
                dst_ref=comm.at[h, j],
                send_sem=rs_send.at[h, j],
                recv_sem=rs_recv.at[h, j],
                device_id=(ring_nbr(j),),
                device_id_type=pl.DeviceIdType.MESH,
            )

        hop_rdmas = []
        for j in TILE_ORDER:
            sbuf[j] = partial_tile(d, j).astype(jnp.bfloat16)
            r = rs_rdma(0, j)
            r.start()
            hop_rdmas.append(r)

        amax_val = jnp.float32(0.0)
        for h in range(N_DEV - 1):
            last = h == N_DEV - 2
            next_rdmas = []
            for i, j in enumerate(TILE_ORDER):
                if j < NT // 2:
                    c_in = lax.rem(d + 2 * N_DEV - 1 - h, N_DEV)
                else:
                    c_in = lax.rem(d + 1 + h, N_DEV)
                hop_rdmas[i].wait()
                copy(comm.at[h, j], tin)
                acc = tin[...].astype(jnp.float32) + partial_tile(c_in, j)
                if last:
                    acc = jnp.maximum(acc, 0.0)
                    amax_val = jnp.maximum(amax_val, jnp.max(acc))
                sbuf[j] = acc.astype(jnp.bfloat16)
                if not last:
                    r = rs_rdma(h + 1, j)
                    r.start()
                    next_rdmas.append(r)
            hop_rdmas = next_rdmas

        amax_ref[pl.ds(d, 1)] = jnp.full((1, 8, 128), amax_val, jnp.float32)
        sends = []
        for k in range(1, N_DEV):
            peer = lax.rem(d + k, N_DEV)
            r = pltpu.make_async_remote_copy(
                src_ref=amax_ref.at[pl.ds(d, 1)],
                dst_ref=amax_ref.at[pl.ds(d, 1)],
                send_sem=ax_send.at[k - 1],
                recv_sem=ax_recv.at[k - 1],
                device_id=(peer,),
                device_id_type=pl.DeviceIdType.MESH,
            )
            r.start()
            sends.append(r)
        for r in sends:
            r.wait_send()
        for k in range(1, N_DEV):
            src = lax.rem(d + N_DEV - k, N_DEV)
            r = pltpu.make_async_remote_copy(
                src_ref=amax_ref.at[pl.ds(d, 1)],
                dst_ref=amax_ref.at[pl.ds(src, 1)],
                send_sem=ax_send.at[k - 1],
                recv_sem=ax_recv.at[k - 1],
                device_id=(d,),
                device_id_type=pl.DeviceIdType.MESH,
            )
            r.wait_recv()
        amax_g = jnp.maximum(jnp.max(amax_ref[...]), jnp.float32(1e-30))
        scale = amax_g / 448.0
        inv_scale = 448.0 / amax_g

        own_r = lax.rem(d + 1, N_DEV)
        own_l = lax.rem(d + 3, N_DEV)

        def owned_chunk(j):
            return own_r if j < NT // 2 else own_l

        def ag_chunk(h, j):
            if j < NT // 2:
                return lax.rem(d + 1 + N_DEV - h, N_DEV)
            return lax.rem(d + 3 + h, N_DEV)

        def dequant_to_out(c, j):
            copy(gat.at[c, j], qt)
            tin[...] = (qt[...].astype(jnp.float32) * scale).astype(
                jnp.bfloat16)
            copy(tin, out_ref.at[pl.ds(c * CH, CH), pl.ds(j * TN, TN)])

        def ag_rdma(h, j):
            c = ag_chunk(h, j)
            return pltpu.make_async_remote_copy(
                src_ref=gat.at[c, j],
                dst_ref=gat.at[c, j],
                send_sem=ag_send.at[h, j],
                recv_sem=ag_recv.at[h, j],
                device_id=(ring_nbr(j),),
                device_id_type=pl.DeviceIdType.MESH,
            )

        hop_rdmas = []
        for j in TILE_ORDER:
            c = owned_chunk(j)
            v = jnp.minimum(sbuf[j].astype(jnp.float32) * inv_scale, 448.0)
            qt[...] = v.astype(jnp.float8_e4m3fn)
            copy(qt, gat.at[c, j])
            r = ag_rdma(0, j)
            r.start()
            hop_rdmas.append(r)

        for j in TILE_ORDER:
            dequant_to_out(owned_chunk(j), j)

        for h in range(N_DEV - 1):
            last = h == N_DEV - 2
            next_rdmas = []
            for j in TILE_ORDER:
                hop_rdmas[TILE_ORDER.index(j)].wait_recv()
                if not last:
                    r = ag_rdma(h + 1, j)
                    r.start()
                    next_rdmas.append(r)
            for j in TILE_ORDER:
                dequant_to_out(ag_chunk(h + 1, j), j)
            for r in hop_rdmas:
                r.wait_send()
            hop_rdmas = next_rdmas

    out_shapes = [
        jax.ShapeDtypeStruct((M, NCOL), jnp.bfloat16),
        jax.ShapeDtypeStruct((N_DEV - 1, NT, CH, TN), jnp.bfloat16),
        jax.ShapeDtypeStruct((N_DEV, NT, CH, TN), jnp.float8_e4m3fn),
    ]
    out, _, _ = pl.pallas_call(
        body,
        out_shape=out_shapes,
        in_specs=[
            pl.BlockSpec(memory_space=pltpu.MemorySpace.VMEM),
            pl.BlockSpec(memory_space=pltpu.MemorySpace.VMEM),
        ],
        out_specs=[
            pl.BlockSpec(memory_space=pltpu.MemorySpace.HBM),
            pl.BlockSpec(memory_space=pltpu.MemorySpace.HBM),
            pl.BlockSpec(memory_space=pltpu.MemorySpace.HBM),
        ],
        scratch_shapes=[
            pltpu.VMEM((NT, CH, TN), jnp.bfloat16),
            pltpu.VMEM((CH, TN), jnp.bfloat16),
            pltpu.VMEM((CH, TN), jnp.float8_e4m3fn),
            pltpu.VMEM((N_DEV, 8, 128), jnp.float32),
            pltpu.SemaphoreType.DMA((N_DEV - 1, NT)),
            pltpu.SemaphoreType.DMA((N_DEV - 1, NT)),
            pltpu.SemaphoreType.DMA((N_DEV - 1,)),
            pltpu.SemaphoreType.DMA((N_DEV - 1,)),
            pltpu.SemaphoreType.DMA((N_DEV - 1, NT)),
            pltpu.SemaphoreType.DMA((N_DEV - 1, NT)),
            pltpu.SemaphoreType.DMA,
        ],
        compiler_params=pltpu.CompilerParams(
            collective_id=0, vmem_limit_bytes=64 * 1024 * 1024),
    )(x, w_mat)
    return out


# device time: 512142 ns/iter; 2.2004x vs baseline; 1.0366x over previous
import jax
import jax.numpy as jnp
from jax import lax
from jax.experimental import pallas as pl
from jax.experimental.pallas import tpu as pltpu

N_DEV = 4
M = 4096
KSH = 1024
NCOL = 8192
CH = M // N_DEV
TN = 1024
NT = NCOL // TN
TILE_ORDER = (0, 4, 1, 5, 2, 6, 3, 7)


def kernel(x, w_mat):
    x = x.astype(jnp.bfloat16)
    w_mat = w_mat.astype(jnp.bfloat16)

    def body(x_ref, w_ref, out_ref, comm, gat, sbuf, tin, qt, amax_ref,
             rs_send, rs_recv, ax_send, ax_recv, ag_send, ag_recv, cp_sem):
        d = lax.axis_index("i")
        left = lax.rem(d + N_DEV - 1, N_DEV)
        right = lax.rem(d + 1, N_DEV)

        def ring_nbr(j):
            return right if j < NT // 2 else left

        barrier = pltpu.get_barrier_semaphore()
        for nbr in (left, right):
            pl.semaphore_signal(barrier, inc=1, device_id=(nbr,),
                                device_id_type=pl.DeviceIdType.MESH)
        pl.semaphore_wait(barrier, 2)

        def partial_tile(c, j):
            xc = x_ref[pl.ds(c * CH, CH), :]
            return jnp.dot(xc, w_ref[:, j * TN:(j + 1) * TN],
                           preferred_element_type=jnp.float32)

        def copy(src, dst):
            cp = pltpu.make_async_copy(src, dst, cp_sem)
            cp.start()
            cp.wait()

        def rs_rdma(h, j):
            return pltpu.make_async_remote_copy(
                src_ref=sbuf.at[j],
                dst_ref=comm.at[h, j],
                send_sem=rs_send.at[h, j],
                recv_sem=rs_recv.at[h, j],
                device_id=(ring_nbr(j),),
                device_id_type=pl.DeviceIdType.MESH,
            )

        hop_rdmas = []
        for j in TILE_ORDER:
            sbuf[j] = partial_tile(d, j).astype(jnp.bfloat16)
            r = rs_rdma(0, j)
            r.start()
            hop_rdmas.append(r)

        amax_val = jnp.float32(0.0)
        for h in range(N_DEV - 1):
            last = h == N_DEV - 2
            next_rdmas = []
            for i, j in enumerate(TILE_ORDER):
                if j < NT // 2:
                    c_in = lax.rem(d + 2 * N_DEV - 1 - h, N_DEV)
                else:
                    c_in = lax.rem(d + 1 + h, N_DEV)
                hop_rdmas[i].wait()
                copy(comm.at[h, j], tin)
                acc = tin[...].astype(jnp.float32) + partial_tile(c_in, j)
                if last:
                    acc = jnp.maximum(acc, 0.0)
                    amax_val = jnp.maximum(amax_val, jnp.max(acc))
                sbuf[j] = acc.astype(jnp.bfloat16)
                if not last:
                    r = rs_rdma(h + 1, j)
                    r.start()
                    next_rdmas.append(r)
            hop_rdmas = next_rdmas

        amax_ref[pl.ds(d, 1)] = jnp.full((1, 8, 128), amax_val, jnp.float32)
        sends = []
        for k in range(1, N_DEV):
            peer = lax.rem(d + k, N_DEV)
            r = pltpu.make_async_remote_copy(
                src_ref=amax_ref.at[pl.ds(d, 1)],
                dst_ref=amax_ref.at[pl.ds(d, 1)],
                send_sem=ax_send.at[k - 1],
                recv_sem=ax_recv.at[k - 1],
                device_id=(peer,),
                device_id_type=pl.DeviceIdType.MESH,
            )
            r.start()
            sends.append(r)
        for r in sends:
            r.wait_send()
        for k in range(1, N_DEV):
            src = lax.rem(d + N_DEV - k, N_DEV)
            r = pltpu.make_async_remote_copy(
                src_ref=amax_ref.at[pl.ds(d, 1)],
                dst_ref=amax_ref.at[pl.ds(src, 1)],
                send_sem=ax_send.at[k - 1],
                recv_sem=ax_recv.at[k - 1],
                device_id=(d,),
                device_id_type=pl.DeviceIdType.MESH,
            )
            r.wait_recv()
        amax_g = jnp.maximum(jnp.max(amax_ref[...]), jnp.float32(1e-30))
        scale = amax_g / 448.0
        inv_scale = 448.0 / amax_g

        own_r = lax.rem(d + 1, N_DEV)
        own_l = lax.rem(d + 3, N_DEV)

        def owned_chunk(j):
            return own_r if j < NT // 2 else own_l

        def ag_chunk(h, j):
            if j < NT // 2:
                return lax.rem(d + 1 + N_DEV - h, N_DEV)
            return lax.rem(d + 3 + h, N_DEV)

        def dequant_to_out(c, j):
            copy(gat.at[c, j], qt)
            tin[...] = (qt[...].astype(jnp.float32) * scale).astype(
                jnp.bfloat16)
            copy(tin, out_ref.at[pl.ds(c * CH, CH), pl.ds(j * TN, TN)])

        def ag_rdma(h, j):
            c = ag_chunk(h, j)
            return pltpu.make_async_remote_copy(
                src_ref=gat.at[c, j],
                dst_ref=gat.at[c, j],
                send_sem=ag_send.at[h, j],
                recv_sem=ag_recv.at[h, j],
                device_id=(ring_nbr(j),),
                device_id_type=pl.DeviceIdType.MESH,
            )

        hop_rdmas = []
        for j in TILE_ORDER:
            c = owned_chunk(j)
            v = jnp.minimum(sbuf[j].astype(jnp.float32) * inv_scale, 448.0)
            qt[...] = v.astype(jnp.float8_e4m3fn)
            copy(qt, gat.at[c, j])
            r = ag_rdma(0, j)
            r.start()
            hop_rdmas.append(r)

        for j in TILE_ORDER:
            dequant_to_out(owned_chunk(j), j)

        for h in range(N_DEV - 1):
            last = h == N_DEV - 2
            next_rdmas = []
            for i, j in enumerate(TILE_ORDER):
                hop_rdmas[i].wait_recv()
                if not last:
                    r = ag_rdma(h + 1, j)
                    r.start()
                    next_rdmas.append(r)
                dequant_to_out(ag_chunk(h + 1, j), j)
            for r in hop_rdmas:
                r.wait_send()
            hop_rdmas = next_rdmas

    out_shapes = [
        jax.ShapeDtypeStruct((M, NCOL), jnp.bfloat16),
        jax.ShapeDtypeStruct((N_DEV - 1, NT, CH, TN), jnp.bfloat16),
        jax.ShapeDtypeStruct((N_DEV, NT, CH, TN), jnp.float8_e4m3fn),
    ]
    out, _, _ = pl.pallas_call(
        body,
        out_shape=out_shapes,
        in_specs=[
            pl.BlockSpec(memory_space=pltpu.MemorySpace.VMEM),
            pl.BlockSpec(memory_space=pltpu.MemorySpace.VMEM),
        ],
        out_specs=[
            pl.BlockSpec(memory_space=pltpu.MemorySpace.HBM),
            pl.BlockSpec(memory_space=pltpu.MemorySpace.HBM),
            pl.BlockSpec(memory_space=pltpu.MemorySpace.HBM),
        ],
        scratch_shapes=[
            pltpu.VMEM((NT, CH, TN), jnp.bfloat16),
            pltpu.VMEM((CH, TN), jnp.bfloat16),
            pltpu.VMEM((CH, TN), jnp.float8_e4m3fn),
            pltpu.VMEM((N_DEV, 8, 128), jnp.float32),
            pltpu.SemaphoreType.DMA((N_DEV - 1, NT)),
            pltpu.SemaphoreType.DMA((N_DEV - 1, NT)),
            pltpu.SemaphoreType.DMA((N_DEV - 1,)),
            pltpu.SemaphoreType.DMA((N_DEV - 1,)),
            pltpu.SemaphoreType.DMA((N_DEV - 1, NT)),
            pltpu.SemaphoreType.DMA((N_DEV - 1, NT)),
            pltpu.SemaphoreType.DMA,
        ],
        compiler_params=pltpu.CompilerParams(
            collective_id=0, vmem_limit_bytes=64 * 1024 * 1024),
    )(x, w_mat)
    return out


# device time: 510058 ns/iter; 2.2094x vs baseline; 1.0041x over previous
import jax
import jax.numpy as jnp
from jax import lax
from jax.experimental import pallas as pl
from jax.experimental.pallas import tpu as pltpu

N_DEV = 4
M = 4096
KSH = 1024
NCOL = 8192
CH = M // N_DEV
TN = 1024
NT = NCOL // TN
TILE_ORDER = (0, 4, 1, 5, 2, 6, 3, 7)


def kernel(x, w_mat):
    x = x.astype(jnp.bfloat16)
    w_mat = w_mat.astype(jnp.bfloat16)

    def body(x_ref, w_ref, out_ref, comm, gat, sbuf, tin, qt, amax_ref,
             rs_send, rs_recv, ax_send, ax_recv, ag_send, ag_recv, cp_sem):
        d = lax.axis_index("i")
        left = lax.rem(d + N_DEV - 1, N_DEV)
        right = lax.rem(d + 1, N_DEV)

        def ring_nbr(j):
            return right if j < NT // 2 else left

        barrier = pltpu.get_barrier_semaphore()
        for nbr in (left, right):
            pl.semaphore_signal(barrier, inc=1, device_id=(nbr,),
                                device_id_type=pl.DeviceIdType.MESH)
        pl.semaphore_wait(barrier, 2)

        def partial_tile(c, j):
            xc = x_ref[pl.ds(c * CH, CH), :]
            return jnp.dot(xc, w_ref[:, j * TN:(j + 1) * TN],
                           preferred_element_type=jnp.float32)

        def copy(src, dst):
            cp = pltpu.make_async_copy(src, dst, cp_sem)
            cp.start()
            cp.wait()

        def rs_rdma(h, j):
            return pltpu.make_async_remote_copy(
                src_ref=sbuf.at[j],
                dst_ref=comm.at[h, j],
                send_sem=rs_send.at[h, j],
                recv_sem=rs_recv.at[h, j],
                device_id=(ring_nbr(j),),
                device_id_type=pl.DeviceIdType.MESH,
            )

        hop_rdmas = []
        for j in TILE_ORDER:
            sbuf[j] = partial_tile(d, j).astype(jnp.bfloat16)
            r = rs_rdma(0, j)
            r.start()
            hop_rdmas.append(r)

        amax_val = jnp.float32(0.0)
        for h in range(N_DEV - 1):
            last = h == N_DEV - 2
            next_rdmas = []
            for i, j in enumerate(TILE_ORDER):
                if j < NT // 2:
                    c_in = lax.rem(d + 2 * N_DEV - 1 - h, N_DEV)
                else:
                    c_in = lax.rem(d + 1 + h, N_DEV)
                part = partial_tile(c_in, j)
                hop_rdmas[i].wait()
                copy(comm.at[h, j], tin)
                acc = tin[...].astype(jnp.float32) + part
                if last:
                    acc = jnp.maximum(acc, 0.0)
                    amax_val = jnp.maximum(amax_val, jnp.max(acc))
                sbuf[j] = acc.astype(jnp.bfloat16)
                if not last:
                    r = rs_rdma(h + 1, j)
                    r.start()
                    next_rdmas.append(r)
            hop_rdmas = next_rdmas

        amax_ref[pl.ds(d, 1)] = jnp.full((1, 8, 128), amax_val, jnp.float32)
        sends = []
        for k in range(1, N_DEV):
            peer = lax.rem(d + k, N_DEV)
            r = pltpu.make_async_remote_copy(
                src_ref=amax_ref.at[pl.ds(d, 1)],
                dst_ref=amax_ref.at[pl.ds(d, 1)],
                send_sem=ax_send.at[k - 1],
                recv_sem=ax_recv.at[k - 1],
                device_id=(peer,),
                device_id_type=pl.DeviceIdType.MESH,
            )
            r.start()
            sends.append(r)
        for r in sends:
            r.wait_send()
        for k in range(1, N_DEV):
            src = lax.rem(d + N_DEV - k, N_DEV)
            r = pltpu.make_async_remote_copy(
                src_ref=amax_ref.at[pl.ds(d, 1)],
                dst_ref=amax_ref.at[pl.ds(src, 1)],
                send_sem=ax_send.at[k - 1],
                recv_sem=ax_recv.at[k - 1],
                device_id=(d,),
                device_id_type=pl.DeviceIdType.MESH,
            )
            r.wait_recv()
        amax_g = jnp.maximum(jnp.max(amax_ref[...]), jnp.float32(1e-30))
        scale = amax_g / 448.0
        inv_scale = 448.0 / amax_g

        own_r = lax.rem(d + 1, N_DEV)
        own_l = lax.rem(d + 3, N_DEV)

        def owned_chunk(j):
            return own_r if j < NT // 2 else own_l

        def ag_chunk(h, j):
            if j < NT // 2:
                return lax.rem(d + 1 + N_DEV - h, N_DEV)
            return lax.rem(d + 3 + h, N_DEV)

        def dequant_to_out(c, j):
            copy(gat.at[c, j], qt)
            tin[...] = (qt[...].astype(jnp.float32) * scale).astype(
                jnp.bfloat16)
            copy(tin, out_ref.at[pl.ds(c * CH, CH), pl.ds(j * TN, TN)])

        def ag_rdma(h, j):
            c = ag_chunk(h, j)
            return pltpu.make_async_remote_copy(
                src_ref=gat.at[c, j],
                dst_ref=gat.at[c, j],
                send_sem=ag_send.at[h, j],
                recv_sem=ag_recv.at[h, j],
                device_id=(ring_nbr(j),),
                device_id_type=pl.DeviceIdType.MESH,
            )

        hop_rdmas = []
        for j in TILE_ORDER:
            c = owned_chunk(j)
            v = jnp.minimum(sbuf[j].astype(jnp.float32) * inv_scale, 448.0)
            qt[...] = v.astype(jnp.float8_e4m3fn)
            copy(qt, gat.at[c, j])
            r = ag_rdma(0, j)
            r.start()
            hop_rdmas.append(r)

        for j in TILE_ORDER:
            dequant_to_out(owned_chunk(j), j)

        for h in range(N_DEV - 1):
            last = h == N_DEV - 2
            next_rdmas = []
            for i, j in enumerate(TILE_ORDER):
                hop_rdmas[i].wait_recv()
                if not last:
                    r = ag_rdma(h + 1, j)
                    r.start()
                    next_rdmas.append(r)
                dequant_to_out(ag_chunk(h + 1, j), j)
            for r in hop_rdmas:
                r.wait_send()
            hop_rdmas = next_rdmas

    out_shapes = [
        jax.ShapeDtypeStruct((M, NCOL), jnp.bfloat16),
        jax.ShapeDtypeStruct((N_DEV - 1, NT, CH, TN), jnp.bfloat16),
        jax.ShapeDtypeStruct((N_DEV, NT, CH, TN), jnp.float8_e4m3fn),
    ]
    out, _, _ = pl.pallas_call(
        body,
        out_shape=out_shapes,
        in_specs=[
            pl.BlockSpec(memory_space=pltpu.MemorySpace.VMEM),
            pl.BlockSpec(memory_space=pltpu.MemorySpace.VMEM),
        ],
        out_specs=[
            pl.BlockSpec(memory_space=pltpu.MemorySpace.HBM),
            pl.BlockSpec(memory_space=pltpu.MemorySpace.HBM),
            pl.BlockSpec(memory_space=pltpu.MemorySpace.HBM),
        ],
        scratch_shapes=[
            pltpu.VMEM((NT, CH, TN), jnp.bfloat16),
            pltpu.VMEM((CH, TN), jnp.bfloat16),
            pltpu.VMEM((CH, TN), jnp.float8_e4m3fn),
            pltpu.VMEM((N_DEV, 8, 128), jnp.float32),
            pltpu.SemaphoreType.DMA((N_DEV - 1, NT)),
            pltpu.SemaphoreType.DMA((N_DEV - 1, NT)),
            pltpu.SemaphoreType.DMA((N_DEV - 1,)),
            pltpu.SemaphoreType.DMA((N_DEV - 1,)),
            pltpu.SemaphoreType.DMA((N_DEV - 1, NT)),
            pltpu.SemaphoreType.DMA((N_DEV - 1, NT)),
            pltpu.SemaphoreType.DMA,
        ],
        compiler_params=pltpu.CompilerParams(
            collective_id=0, vmem_limit_bytes=64 * 1024 * 1024),
    )(x, w_mat)
    return out
